# Initial kernel scaffold; baseline (speedup 1.0000x reference)
import jax
import jax.numpy as jnp
from jax.experimental import pallas as pl


def kernel(X, E, emb_nodes, emb_edges, edge_index, edge_W, edge_b, node_W, node_b):
    raise NotImplementedError("write your pallas kernel here")


# speedup vs baseline: 2.2436x; 2.2436x over previous
import functools

import jax
import jax.numpy as jnp
from jax import lax
from jax.experimental import pallas as pl
from jax.experimental.pallas import tpu as pltpu
from jax.experimental.pallas import tpu_sc as plsc

F32 = jnp.float32
NC = 2
NS = 16
LANES = 16


def _proj_body(x_ref, w_ref, a_ref, b_ref):
    x = x_ref[...]
    d = x.shape[1]
    a_ref[...] = jnp.dot(x, w_ref[0:d, :], preferred_element_type=F32)
    b_ref[...] = jnp.dot(x, w_ref[d:2 * d, :], preferred_element_type=F32)


def _proj(X, edge_W, blk):
    N, D = X.shape
    return pl.pallas_call(
        _proj_body,
        grid=(N // blk,),
        in_specs=[
            pl.BlockSpec((blk, D), lambda i: (i, 0)),
            pl.BlockSpec((3 * D, D), lambda i: (0, 0)),
        ],
        out_specs=[
            pl.BlockSpec((blk, D), lambda i: (i, 0)),
            pl.BlockSpec((blk, D), lambda i: (i, 0)),
        ],
        out_shape=[
            jax.ShapeDtypeStruct((N, D), F32),
            jax.ShapeDtypeStruct((N, D), F32),
        ],
    )(X, edge_W)


def _edge_body(e_ref, gm_ref, gn_ref, w3_ref, b_ref, msg_ref, enew_ref):
    e = e_ref[...]
    c = jnp.dot(e, w3_ref[...], preferred_element_type=F32) + b_ref[...]
    pm = gm_ref[...] + c
    pn = gn_ref[...] + c
    msg_ref[...] = jnp.where(pm > 0, pm, jnp.exp(pm) - 1.0)
    enew_ref[...] = jnp.where(pn > 0, pn, jnp.exp(pn) - 1.0) + e


def _edge_mlp(E, Gmsg, Gnew, W3, b2d, blk):
    En, D = E.shape
    blk_spec = pl.BlockSpec((blk, D), lambda i: (i, 0))
    return pl.pallas_call(
        _edge_body,
        grid=(En // blk,),
        in_specs=[
            blk_spec, blk_spec, blk_spec,
            pl.BlockSpec((D, D), lambda i: (0, 0)),
            pl.BlockSpec((1, D), lambda i: (0, 0)),
        ],
        out_specs=[blk_spec, blk_spec],
        out_shape=[
            jax.ShapeDtypeStruct((En, D), F32),
            jax.ShapeDtypeStruct((En, D), F32),
        ],
    )(E, Gmsg, Gnew, W3, b2d)


def _node_body(ag_ref, x_ref, w_ref, b_ref, out_ref):
    x = x_ref[...]
    d = x.shape[1]
    h = (jnp.dot(ag_ref[...], w_ref[0:d, :], preferred_element_type=F32)
         + jnp.dot(x, w_ref[d:2 * d, :], preferred_element_type=F32)
         + b_ref[...])
    out_ref[...] = jnp.where(h > 0, h, jnp.exp(h) - 1.0) + x


def _node_mlp(aggr, X, node_W, b2d, blk):
    N, D = X.shape
    blk_spec = pl.BlockSpec((blk, D), lambda i: (i, 0))
    return pl.pallas_call(
        _node_body,
        grid=(N // blk,),
        in_specs=[
            blk_spec, blk_spec,
            pl.BlockSpec((2 * D, D), lambda i: (0, 0)),
            pl.BlockSpec((1, D), lambda i: (0, 0)),
        ],
        out_specs=blk_spec,
        out_shape=jax.ShapeDtypeStruct((N, D), F32),
    )(aggr, X, node_W, b2d)


def _sc_gather(A, B, row, col, CB):
    N, D = A.shape
    En = row.shape[0]
    NW = NC * NS
    ew = En // NW
    assert En % NW == 0 and ew % CB == 0 and (ew // CB) % 2 == 1
    nchunks = ew // CB
    npairs = (nchunks - 1) // 2
    nsl = D // LANES
    mesh = plsc.VectorSubcoreMesh(core_axis_name="c", subcore_axis_name="s")

    @functools.partial(
        pl.kernel,
        out_type=[
            jax.ShapeDtypeStruct((En, D), F32),
            jax.ShapeDtypeStruct((En, D), F32),
        ],
        mesh=mesh,
        scratch_types=[
            pltpu.VMEM((2, CB, D), F32),
            pltpu.VMEM((2, CB, D), F32),
            pltpu.VMEM((2, CB, D), F32),
            pltpu.VMEM((2, CB, D), F32),
            pltpu.VMEM((2, CB), jnp.int32),
            pltpu.VMEM((2, CB), jnp.int32),
            pltpu.SemaphoreType.DMA,
            pltpu.SemaphoreType.DMA,
            pltpu.SemaphoreType.DMA,
            pltpu.SemaphoreType.DMA,
            pltpu.SemaphoreType.DMA,
            pltpu.SemaphoreType.DMA,
        ],
    )
    def gather_kernel(a_hbm, b_hbm, row_hbm, col_hbm, gmsg_hbm, gnew_hbm,
                      bAc, bBr, bAr, bBc, ixr, ixc,
                      semg0, semg1, semi0, semi1, semw0, semw1):
        wid = lax.axis_index("s") * NC + lax.axis_index("c")
        base = wid * ew
        semg = (semg0, semg1)
        semi = (semi0, semi1)
        semw = (semw0, semw1)

        def fire(t, S):
            eoff = base + t * CB
            cpr = pltpu.async_copy(row_hbm.at[pl.ds(eoff, CB)],
                                   ixr.at[S], semi[S])
            cpc = pltpu.async_copy(col_hbm.at[pl.ds(eoff, CB)],
                                   ixc.at[S], semi[S])
            cpr.wait()
            cpc.wait()
            pltpu.async_copy(a_hbm.at[ixc.at[S]], bAc.at[S], semg[S])
            pltpu.async_copy(b_hbm.at[ixr.at[S]], bBr.at[S], semg[S])
            pltpu.async_copy(a_hbm.at[ixr.at[S]], bAr.at[S], semg[S])
            pltpu.async_copy(b_hbm.at[ixc.at[S]], bBc.at[S], semg[S])

        def finish(t, S):
            pltpu.make_async_copy(a_hbm.at[ixc.at[S]], bAc.at[S],
                                  semg[S]).wait()
            pltpu.make_async_copy(b_hbm.at[ixr.at[S]], bBr.at[S],
                                  semg[S]).wait()
            pltpu.make_async_copy(a_hbm.at[ixr.at[S]], bAr.at[S],
                                  semg[S]).wait()
            pltpu.make_async_copy(b_hbm.at[ixc.at[S]], bBc.at[S],
                                  semg[S]).wait()

            def addrow(i, carry):
                for j in range(nsl):
                    sl = pl.ds(j * LANES, LANES)
                    bAc[S, i, sl] = bAc[S, i, sl] + bBr[S, i, sl]
                    bAr[S, i, sl] = bAr[S, i, sl] + bBc[S, i, sl]
                return carry

            lax.fori_loop(0, CB, addrow, 0, unroll=2)
            eoff = base + t * CB
            w1 = pltpu.async_copy(bAc.at[S], gmsg_hbm.at[pl.ds(eoff, CB)],
                                  semw[S])
            w2 = pltpu.async_copy(bAr.at[S], gnew_hbm.at[pl.ds(eoff, CB)],
                                  semw[S])
            w1.wait()
            w2.wait()

        fire(0, 0)

        def pair_body(p, carry):
            t1 = 1 + 2 * p
            fire(t1, 1)
            finish(t1 - 1, 0)
            fire(t1 + 1, 0)
            finish(t1, 1)
            return carry

        if npairs > 0:
            lax.fori_loop(0, npairs, pair_body, 0)
        finish(nchunks - 1, 0)

    return gather_kernel(A, B, row, col)


def _sc_scatter(msg, col, N, CB):
    En, D = msg.shape
    DH = D // NC
    ept = En // NS
    assert En % NS == 0 and ept % CB == 0 and (ept // CB) % 2 == 1
    nchunks = ept // CB
    npairs = (nchunks - 1) // 2
    RB = 80
    assert N % RB == 0
    nrb = N // RB
    mesh = plsc.VectorSubcoreMesh(core_axis_name="c", subcore_axis_name="s")

    @functools.partial(
        pl.kernel,
        out_type=jax.ShapeDtypeStruct((N, D), F32),
        mesh=mesh,
        scratch_types=[
            pltpu.VMEM_SHARED((N, DH), F32),
            pltpu.VMEM((2, CB, DH), F32),
            pltpu.VMEM((2, CB), jnp.int32),
            pltpu.VMEM((RB, DH), F32),
            pltpu.SemaphoreType.DMA,
            pltpu.SemaphoreType.DMA,
        ],
    )
    def scatter_kernel(msg_hbm, col_hbm, out_hbm, acc, mbuf, ixc, zbuf,
                       sem0, sem1):
        c = lax.axis_index("c")
        s = lax.axis_index("s")
        sems = (sem0, sem1)
        coff = c * DH

        def zrow(i, carry):
            for j in range(DH // LANES):
                zbuf[i, pl.ds(j * LANES, LANES)] = jnp.zeros((LANES,), F32)
            return carry

        lax.fori_loop(0, RB, zrow, 0)
        nblk = (nrb - s + NS - 1) // NS

        def zblk(k, carry):
            roff = (s + k * NS) * RB
            pltpu.sync_copy(zbuf, acc.at[pl.ds(roff, RB)])
            return carry

        lax.fori_loop(0, nblk, zblk, 0)
        plsc.subcore_barrier()

        def fire(t, S):
            eoff = s * ept + t * CB
            pltpu.async_copy(col_hbm.at[pl.ds(eoff, CB)], ixc.at[S],
                             sems[S])
            pltpu.async_copy(msg_hbm.at[pl.ds(eoff, CB), pl.ds(coff, DH)],
                             mbuf.at[S], sems[S])

        def finish(t, S):
            eoff = s * ept + t * CB
            pltpu.make_async_copy(col_hbm.at[pl.ds(eoff, CB)], ixc.at[S],
                                  sems[S]).wait()
            pltpu.make_async_copy(
                msg_hbm.at[pl.ds(eoff, CB), pl.ds(coff, DH)],
                mbuf.at[S], sems[S]).wait()
            pltpu.sync_copy(mbuf.at[S], acc.at[ixc.at[S]], add=True)

        fire(0, 0)

        def pair_body(p, carry):
            t1 = 1 + 2 * p
            fire(t1, 1)
            finish(t1 - 1, 0)
            fire(t1 + 1, 0)
            finish(t1, 1)
            return carry

        if npairs > 0:
            lax.fori_loop(0, npairs, pair_body, 0)
        finish(nchunks - 1, 0)

        plsc.subcore_barrier()

        def wblk(k, carry):
            roff = (s + k * NS) * RB
            pltpu.sync_copy(acc.at[pl.ds(roff, RB)], zbuf)
            pltpu.sync_copy(zbuf, out_hbm.at[pl.ds(roff, RB),
                                             pl.ds(coff, DH)])
            return carry

        lax.fori_loop(0, nblk, wblk, 0)

    return scatter_kernel(msg, col)


def kernel(X, E, emb_nodes, emb_edges, edge_index, edge_W, edge_b,
           node_W, node_b):
    N, D = X.shape
    En = E.shape[0]
    row = edge_index[0]
    col = edge_index[1]
    eb2 = edge_b.reshape(1, D)
    nb2 = node_b.reshape(1, D)
    W3 = lax.slice(edge_W, (2 * D, 0), (3 * D, D))

    A, B = _proj(X, edge_W, blk=1000)
    Gmsg, Gnew = _sc_gather(A, B, row, col, CB=40)
    msg, E_new = _edge_mlp(E, Gmsg, Gnew, W3, eb2, blk=1000)
    aggr = _sc_scatter(msg, col, N, CB=80)
    X_new = _node_mlp(aggr, X, node_W, nb2, blk=1000)
    return X_new, E_new


# speedup vs baseline: 2.4889x; 1.1093x over previous
import functools

import jax
import jax.numpy as jnp
from jax import lax
from jax.experimental import pallas as pl
from jax.experimental.pallas import tpu as pltpu
from jax.experimental.pallas import tpu_sc as plsc

F32 = jnp.float32
NC = 2
NS = 16
LANES = 16


def _proj_body(x_ref, w_ref, p_ref):
    x = x_ref[...]
    d = x.shape[1]
    a = jnp.dot(x, w_ref[0:d, :], preferred_element_type=F32)
    b = jnp.dot(x, w_ref[d:2 * d, :], preferred_element_type=F32)
    a32 = lax.bitcast_convert_type(
        a.astype(jnp.bfloat16), jnp.uint16).astype(jnp.uint32)
    b32 = lax.bitcast_convert_type(
        b.astype(jnp.bfloat16), jnp.uint16).astype(jnp.uint32)
    p_ref[...] = lax.bitcast_convert_type(
        a32 | (b32 << jnp.uint32(16)), jnp.int32)


def _proj(X, edge_W, blk):
    N, D = X.shape
    return pl.pallas_call(
        _proj_body,
        grid=(N // blk,),
        in_specs=[
            pl.BlockSpec((blk, D), lambda i: (i, 0)),
            pl.BlockSpec((3 * D, D), lambda i: (0, 0)),
        ],
        out_specs=pl.BlockSpec((blk, D), lambda i: (i, 0)),
        out_shape=jax.ShapeDtypeStruct((N, D), jnp.int32),
    )(X, edge_W)


def _edge_body(e_ref, g_ref, w3_ref, b_ref, msg_ref, enew_ref):
    e = e_ref[...]
    c = jnp.dot(e, w3_ref[...], preferred_element_type=F32) + b_ref[...]
    gi = g_ref[...]
    gm = lax.bitcast_convert_type(
        (gi & 0xFFFF).astype(jnp.uint16), jnp.bfloat16).astype(F32)
    gn = lax.bitcast_convert_type(
        lax.shift_right_logical(gi, 16).astype(jnp.uint16),
        jnp.bfloat16).astype(F32)
    pm = gm + c
    pn = gn + c
    msg_ref[...] = jnp.where(pm > 0, pm, jnp.exp(pm) - 1.0)
    enew_ref[...] = jnp.where(pn > 0, pn, jnp.exp(pn) - 1.0) + e


def _edge_mlp(E, G, W3, b2d, blk):
    En, D = E.shape
    blk_spec = pl.BlockSpec((blk, D), lambda i: (i, 0))
    return pl.pallas_call(
        _edge_body,
        grid=(En // blk,),
        in_specs=[
            blk_spec, blk_spec,
            pl.BlockSpec((D, D), lambda i: (0, 0)),
            pl.BlockSpec((1, D), lambda i: (0, 0)),
        ],
        out_specs=[blk_spec, blk_spec],
        out_shape=[
            jax.ShapeDtypeStruct((En, D), F32),
            jax.ShapeDtypeStruct((En, D), F32),
        ],
    )(E, G, W3, b2d)


def _node_body(ag_ref, x_ref, w_ref, b_ref, out_ref):
    x = x_ref[...]
    d = x.shape[1]
    h = (jnp.dot(ag_ref[...], w_ref[0:d, :], preferred_element_type=F32)
         + jnp.dot(x, w_ref[d:2 * d, :], preferred_element_type=F32)
         + b_ref[...])
    out_ref[...] = jnp.where(h > 0, h, jnp.exp(h) - 1.0) + x


def _node_mlp(aggr, X, node_W, b2d, blk):
    N, D = X.shape
    blk_spec = pl.BlockSpec((blk, D), lambda i: (i, 0))
    return pl.pallas_call(
        _node_body,
        grid=(N // blk,),
        in_specs=[
            blk_spec, blk_spec,
            pl.BlockSpec((2 * D, D), lambda i: (0, 0)),
            pl.BlockSpec((1, D), lambda i: (0, 0)),
        ],
        out_specs=blk_spec,
        out_shape=jax.ShapeDtypeStruct((N, D), F32),
    )(aggr, X, node_W, b2d)


def _sc_gather(P, row, col, CB):
    N, D = P.shape
    En = row.shape[0]
    NW = NC * NS
    ew = En // NW
    assert En % NW == 0 and ew % CB == 0 and (ew // CB) % 2 == 1
    nchunks = ew // CB
    npairs = (nchunks - 1) // 2
    nsl = D // LANES
    mesh = plsc.VectorSubcoreMesh(core_axis_name="c", subcore_axis_name="s")

    @functools.partial(
        pl.kernel,
        out_type=jax.ShapeDtypeStruct((En, D), jnp.int32),
        mesh=mesh,
        scratch_types=[
            pltpu.VMEM((2, CB, D), jnp.int32),
            pltpu.VMEM((2, CB, D), jnp.int32),
            pltpu.VMEM((2, CB), jnp.int32),
            pltpu.VMEM((2, CB), jnp.int32),
            pltpu.SemaphoreType.DMA,
            pltpu.SemaphoreType.DMA,
            pltpu.SemaphoreType.DMA,
            pltpu.SemaphoreType.DMA,
            pltpu.SemaphoreType.DMA,
            pltpu.SemaphoreType.DMA,
        ],
    )
    def gather_kernel(p_hbm, row_hbm, col_hbm, g_hbm,
                      bPc, bPr, ixr, ixc,
                      semg0, semg1, semi0, semi1, semw0, semw1):
        wid = lax.axis_index("s") * NC + lax.axis_index("c")
        base = wid * ew
        semg = (semg0, semg1)
        semi = (semi0, semi1)
        semw = (semw0, semw1)

        def fire(t, S):
            eoff = base + t * CB
            cpr = pltpu.async_copy(row_hbm.at[pl.ds(eoff, CB)],
                                   ixr.at[S], semi[S])
            cpc = pltpu.async_copy(col_hbm.at[pl.ds(eoff, CB)],
                                   ixc.at[S], semi[S])
            cpr.wait()
            cpc.wait()
            pltpu.async_copy(p_hbm.at[ixc.at[S]], bPc.at[S], semg[S])
            pltpu.async_copy(p_hbm.at[ixr.at[S]], bPr.at[S], semg[S])

        def finish(t, S):
            pltpu.make_async_copy(p_hbm.at[ixc.at[S]], bPc.at[S],
                                  semg[S]).wait()
            pltpu.make_async_copy(p_hbm.at[ixr.at[S]], bPr.at[S],
                                  semg[S]).wait()

            def addrow(i, carry):
                for j in range(nsl):
                    sl = pl.ds(j * LANES, LANES)
                    bc_ = lax.bitcast_convert_type
                    cv = bc_(bPc[S, i, sl], jnp.uint32)
                    rv = bc_(bPr[S, i, sl], jnp.uint32)
                    hi = jnp.uint32(0xFFFF0000)
                    ac = bc_(cv << jnp.uint32(16), F32)
                    bc = bc_(cv & hi, F32)
                    ar = bc_(rv << jnp.uint32(16), F32)
                    br = bc_(rv & hi, F32)
                    gm = bc_(ac + br, jnp.uint32)
                    gn = bc_(ar + bc, jnp.uint32)
                    gmb = (gm + jnp.uint32(0x8000)) >> jnp.uint32(16)
                    gnb = (gn + jnp.uint32(0x8000)) & hi
                    bPc[S, i, sl] = bc_(gmb | gnb, jnp.int32)
                return carry

            lax.fori_loop(0, CB, addrow, 0, unroll=2)
            eoff = base + t * CB
            pltpu.async_copy(bPc.at[S], g_hbm.at[pl.ds(eoff, CB)],
                             semw[S]).wait()

        fire(0, 0)

        def pair_body(p, carry):
            t1 = 1 + 2 * p
            fire(t1, 1)
            finish(t1 - 1, 0)
            fire(t1 + 1, 0)
            finish(t1, 1)
            return carry

        if npairs > 0:
            lax.fori_loop(0, npairs, pair_body, 0)
        finish(nchunks - 1, 0)

    return gather_kernel(P, row, col)


def _sc_scatter(msg, col, N, CB):
    En, D = msg.shape
    DH = D // NC
    ept = En // NS
    assert En % NS == 0 and ept % CB == 0 and (ept // CB) % 2 == 1
    nchunks = ept // CB
    npairs = (nchunks - 1) // 2
    RB = 80
    assert N % RB == 0
    nrb = N // RB
    mesh = plsc.VectorSubcoreMesh(core_axis_name="c", subcore_axis_name="s")

    @functools.partial(
        pl.kernel,
        out_type=jax.ShapeDtypeStruct((N, D), F32),
        mesh=mesh,
        scratch_types=[
            pltpu.VMEM_SHARED((N, DH), F32),
            pltpu.VMEM((2, CB, DH), F32),
            pltpu.VMEM((2, CB), jnp.int32),
            pltpu.VMEM((RB, DH), F32),
            pltpu.SemaphoreType.DMA,
            pltpu.SemaphoreType.DMA,
        ],
    )
    def scatter_kernel(msg_hbm, col_hbm, out_hbm, acc, mbuf, ixc, zbuf,
                       sem0, sem1):
        c = lax.axis_index("c")
        s = lax.axis_index("s")
        sems = (sem0, sem1)
        coff = c * DH

        def zrow(i, carry):
            for j in range(DH // LANES):
                zbuf[i, pl.ds(j * LANES, LANES)] = jnp.zeros((LANES,), F32)
            return carry

        lax.fori_loop(0, RB, zrow, 0)
        nblk = (nrb - s + NS - 1) // NS

        def zblk(k, carry):
            roff = (s + k * NS) * RB
            pltpu.sync_copy(zbuf, acc.at[pl.ds(roff, RB)])
            return carry

        lax.fori_loop(0, nblk, zblk, 0)
        plsc.subcore_barrier()

        def fire(t, S):
            eoff = s * ept + t * CB
            pltpu.async_copy(col_hbm.at[pl.ds(eoff, CB)], ixc.at[S],
                             sems[S])
            pltpu.async_copy(msg_hbm.at[pl.ds(eoff, CB), pl.ds(coff, DH)],
                             mbuf.at[S], sems[S])

        def finish(t, S):
            eoff = s * ept + t * CB
            pltpu.make_async_copy(col_hbm.at[pl.ds(eoff, CB)], ixc.at[S],
                                  sems[S]).wait()
            pltpu.make_async_copy(
                msg_hbm.at[pl.ds(eoff, CB), pl.ds(coff, DH)],
                mbuf.at[S], sems[S]).wait()
            pltpu.sync_copy(mbuf.at[S], acc.at[ixc.at[S]], add=True)

        fire(0, 0)

        def pair_body(p, carry):
            t1 = 1 + 2 * p
            fire(t1, 1)
            finish(t1 - 1, 0)
            fire(t1 + 1, 0)
            finish(t1, 1)
            return carry

        if npairs > 0:
            lax.fori_loop(0, npairs, pair_body, 0)
        finish(nchunks - 1, 0)

        plsc.subcore_barrier()

        def wblk(k, carry):
            roff = (s + k * NS) * RB
            pltpu.sync_copy(acc.at[pl.ds(roff, RB)], zbuf)
            pltpu.sync_copy(zbuf, out_hbm.at[pl.ds(roff, RB),
                                             pl.ds(coff, DH)])
            return carry

        lax.fori_loop(0, nblk, wblk, 0)

    return scatter_kernel(msg, col)


def kernel(X, E, emb_nodes, emb_edges, edge_index, edge_W, edge_b,
           node_W, node_b):
    N, D = X.shape
    En = E.shape[0]
    row = edge_index[0]
    col = edge_index[1]
    eb2 = edge_b.reshape(1, D)
    nb2 = node_b.reshape(1, D)
    W3 = lax.slice(edge_W, (2 * D, 0), (3 * D, D))

    P = _proj(X, edge_W, blk=1000)
    G = _sc_gather(P, row, col, CB=40)
    msg, E_new = _edge_mlp(E, G, W3, eb2, blk=1000)
    aggr = _sc_scatter(msg, col, N, CB=80)
    X_new = _node_mlp(aggr, X, node_W, nb2, blk=1000)
    return X_new, E_new
